# baseline (speedup 1.0000x reference)
import numpy as np
import jax
import jax.numpy as jnp
from jax.experimental import pallas as pl
from jax.experimental.pallas import tpu as pltpu

_N = 4096
_BD = 128
_B = 512


def _diag_body(s_ref, d_ref):
    blk = s_ref[...]
    rows = jax.lax.broadcasted_iota(jnp.int32, (_BD, _BD), 0)
    cols = jax.lax.broadcasted_iota(jnp.int32, (_BD, _BD), 1)
    d_ref[0, :] = jnp.sum(jnp.where(rows == cols, blk, 0.0), axis=0)


def _main_body(im_ref, jm_ref, mu_i_ref, mu_j_ref, d_i_ref, d_j_ref,
               a_ref, b_ref, mud_ref, sd_ref):
    s = pl.program_id(1)

    @pl.when(s == 0)
    def _upper():
        sd = (d_i_ref[0, :][:, None] + d_j_ref[0, :][None, :]
              - a_ref[...] - b_ref[...].T)
        sd_ref[...] = jnp.sqrt(jnp.maximum(sd, 1e-6))
        mud_ref[...] = mu_i_ref[0, :][:, None] - mu_j_ref[0, :][None, :]

    @pl.when(s == 1)
    def _lower():
        sd = (d_j_ref[0, :][:, None] + d_i_ref[0, :][None, :]
              - b_ref[...] - a_ref[...].T)
        sd_ref[...] = jnp.sqrt(jnp.maximum(sd, 1e-6))
        mud_ref[...] = mu_j_ref[0, :][:, None] - mu_i_ref[0, :][None, :]


def kernel(mu, Sigma):
    mu2 = mu.reshape(1, _N)
    nd = _N // _BD
    diag = pl.pallas_call(
        _diag_body,
        grid=(nd,),
        in_specs=[pl.BlockSpec((_BD, _BD), lambda i: (i, i))],
        out_specs=pl.BlockSpec((1, _BD), lambda i: (0, i)),
        out_shape=jax.ShapeDtypeStruct((1, _N), jnp.float32),
    )(Sigma)

    nb = _N // _B
    pairs = [(i, j) for i in range(nb) for j in range(i, nb)]
    imap = jnp.asarray(np.array([p[0] for p in pairs], dtype=np.int32))
    jmap = jnp.asarray(np.array([p[1] for p in pairs], dtype=np.int32))

    def _ti(t, s, im, jm):
        return im[t]

    def _tj(t, s, im, jm):
        return jm[t]

    def _out_r(t, s, im, jm):
        return jnp.where(s == 0, im[t], jm[t])

    def _out_c(t, s, im, jm):
        return jnp.where(s == 0, jm[t], im[t])

    grid_spec = pltpu.PrefetchScalarGridSpec(
        num_scalar_prefetch=2,
        grid=(len(pairs), 2),
        in_specs=[
            pl.BlockSpec((1, _B), lambda t, s, im, jm: (0, _ti(t, s, im, jm))),
            pl.BlockSpec((1, _B), lambda t, s, im, jm: (0, _tj(t, s, im, jm))),
            pl.BlockSpec((1, _B), lambda t, s, im, jm: (0, _ti(t, s, im, jm))),
            pl.BlockSpec((1, _B), lambda t, s, im, jm: (0, _tj(t, s, im, jm))),
            pl.BlockSpec((_B, _B),
                         lambda t, s, im, jm: (_ti(t, s, im, jm),
                                               _tj(t, s, im, jm))),
            pl.BlockSpec((_B, _B),
                         lambda t, s, im, jm: (_tj(t, s, im, jm),
                                               _ti(t, s, im, jm))),
        ],
        out_specs=[
            pl.BlockSpec((_B, _B),
                         lambda t, s, im, jm: (_out_r(t, s, im, jm),
                                               _out_c(t, s, im, jm))),
            pl.BlockSpec((_B, _B),
                         lambda t, s, im, jm: (_out_r(t, s, im, jm),
                                               _out_c(t, s, im, jm))),
        ],
    )
    mud, sd = pl.pallas_call(
        _main_body,
        grid_spec=grid_spec,
        out_shape=[
            jax.ShapeDtypeStruct((_N, _N), jnp.float32),
            jax.ShapeDtypeStruct((_N, _N), jnp.float32),
        ],
    )(imap, jmap, mu2, mu2, diag, diag, Sigma, Sigma)

    return (mud.reshape(_N * _N), sd.reshape(_N * _N))


# speedup vs baseline: 1.2381x; 1.2381x over previous
import numpy as np
import jax
import jax.numpy as jnp
from jax.experimental import pallas as pl
from jax.experimental.pallas import tpu as pltpu

_N = 4096
_BD = 128
_B = 512
_NB = _N // _B
_PAIRS = [(i, j) for i in range(_NB) for j in range(i, _NB)]
_P = len(_PAIRS)


def _diag_body(s_ref, d_ref):
    blk = s_ref[...]
    rows = jax.lax.broadcasted_iota(jnp.int32, (_BD, _BD), 0)
    cols = jax.lax.broadcasted_iota(jnp.int32, (_BD, _BD), 1)
    d_ref[0, :] = jnp.sum(jnp.where(rows == cols, blk, 0.0), axis=0)


def _main_body(im_ref, jm_ref, mu_i_ref, mu_j_ref, d_i_ref, d_j_ref,
               a_ref, b_ref, mud_hbm, sd_hbm,
               mud_up, mud_lo, sd_up, sd_lo, sems):
    t = pl.program_id(0)
    slot = jax.lax.rem(t, 2)
    ti = im_ref[t]
    tj = jm_ref[t]
    r0 = ti * _B
    c0 = tj * _B

    def copies(sl, rr, cc):
        return [
            pltpu.make_async_copy(
                mud_up.at[sl], mud_hbm.at[pl.ds(rr, _B), pl.ds(cc, _B)],
                sems.at[sl]),
            pltpu.make_async_copy(
                mud_lo.at[sl], mud_hbm.at[pl.ds(cc, _B), pl.ds(rr, _B)],
                sems.at[sl]),
            pltpu.make_async_copy(
                sd_up.at[sl], sd_hbm.at[pl.ds(rr, _B), pl.ds(cc, _B)],
                sems.at[sl]),
            pltpu.make_async_copy(
                sd_lo.at[sl], sd_hbm.at[pl.ds(cc, _B), pl.ds(rr, _B)],
                sems.at[sl]),
        ]

    @pl.when(t >= 2)
    def _drain_prev():
        for c in copies(slot, r0, c0):
            c.wait()

    a = a_ref[...]
    bt = b_ref[...].T
    dsum = d_i_ref[0, :][:, None] + d_j_ref[0, :][None, :]
    sdv = jnp.sqrt(jnp.maximum(dsum - a - bt, 1e-6))
    mudv = mu_i_ref[0, :][:, None] - mu_j_ref[0, :][None, :]
    sd_up[slot] = sdv
    sd_lo[slot] = sdv.T
    mud_up[slot] = mudv
    mud_lo[slot] = -mudv.T

    for c in copies(slot, r0, c0):
        c.start()

    @pl.when(t == _P - 1)
    def _drain_tail():
        for c in copies(1 - slot, r0, c0):
            c.wait()
        for c in copies(slot, r0, c0):
            c.wait()


def kernel(mu, Sigma):
    mu2 = mu.reshape(1, _N)
    nd = _N // _BD
    diag = pl.pallas_call(
        _diag_body,
        grid=(nd,),
        in_specs=[pl.BlockSpec((_BD, _BD), lambda i: (i, i))],
        out_specs=pl.BlockSpec((1, _BD), lambda i: (0, i)),
        out_shape=jax.ShapeDtypeStruct((1, _N), jnp.float32),
    )(Sigma)

    imap = jnp.asarray(np.array([p[0] for p in _PAIRS], dtype=np.int32))
    jmap = jnp.asarray(np.array([p[1] for p in _PAIRS], dtype=np.int32))

    grid_spec = pltpu.PrefetchScalarGridSpec(
        num_scalar_prefetch=2,
        grid=(_P,),
        in_specs=[
            pl.BlockSpec((1, _B), lambda t, im, jm: (0, im[t])),
            pl.BlockSpec((1, _B), lambda t, im, jm: (0, jm[t])),
            pl.BlockSpec((1, _B), lambda t, im, jm: (0, im[t])),
            pl.BlockSpec((1, _B), lambda t, im, jm: (0, jm[t])),
            pl.BlockSpec((_B, _B), lambda t, im, jm: (im[t], jm[t])),
            pl.BlockSpec((_B, _B), lambda t, im, jm: (jm[t], im[t])),
        ],
        out_specs=[
            pl.BlockSpec(memory_space=pl.ANY),
            pl.BlockSpec(memory_space=pl.ANY),
        ],
        scratch_shapes=[
            pltpu.VMEM((2, _B, _B), jnp.float32),
            pltpu.VMEM((2, _B, _B), jnp.float32),
            pltpu.VMEM((2, _B, _B), jnp.float32),
            pltpu.VMEM((2, _B, _B), jnp.float32),
            pltpu.SemaphoreType.DMA((2,)),
        ],
    )
    mud, sd = pl.pallas_call(
        _main_body,
        grid_spec=grid_spec,
        out_shape=[
            jax.ShapeDtypeStruct((_N, _N), jnp.float32),
            jax.ShapeDtypeStruct((_N, _N), jnp.float32),
        ],
    )(imap, jmap, mu2, mu2, diag, diag, Sigma, Sigma)

    return (mud.reshape(_N * _N), sd.reshape(_N * _N))


# speedup vs baseline: 2.6197x; 2.1159x over previous
import numpy as np
import jax
import jax.numpy as jnp
from jax.experimental import pallas as pl
from jax.experimental.pallas import tpu as pltpu

_N = 4096
_BD = 128
_B = 512
_NB = _N // _B
_PAIRS = [(i, j) for i in range(_NB) for j in range(i, _NB)]
_P = len(_PAIRS)


def _diag_body(s_ref, d_ref):
    blk = s_ref[...]
    rows = jax.lax.broadcasted_iota(jnp.int32, (_BD, _BD), 0)
    cols = jax.lax.broadcasted_iota(jnp.int32, (_BD, _BD), 1)
    d_ref[0, :] = jnp.sum(jnp.where(rows == cols, blk, 0.0), axis=0)


def _main_body(im_ref, jm_ref, mu_i_ref, mu_j_ref, d_i_ref, d_j_ref,
               a_ref, b_ref, mud_hbm, sd_hbm,
               mud_up, mud_lo, sd_up, sd_lo, sems):
    t = pl.program_id(0)
    slot = jax.lax.rem(t, 2)
    ti = im_ref[t]
    tj = jm_ref[t]
    r0 = ti * _B
    c0 = tj * _B

    nch = _B // 128

    def copies(sl, rr, cc):
        cs = []
        for kk in range(nch):
            for src, dst in ((mud_up, mud_hbm), (sd_up, sd_hbm)):
                cs.append(pltpu.make_async_copy(
                    src.at[sl, :, pl.ds(kk * 128, 128)],
                    dst.at[pl.ds(rr, _B), cc // 128 + kk, :],
                    sems.at[sl]))
            for src, dst in ((mud_lo, mud_hbm), (sd_lo, sd_hbm)):
                cs.append(pltpu.make_async_copy(
                    src.at[sl, :, pl.ds(kk * 128, 128)],
                    dst.at[pl.ds(cc, _B), rr // 128 + kk, :],
                    sems.at[sl]))
        return cs

    @pl.when(t >= 2)
    def _drain_prev():
        for c in copies(slot, r0, c0):
            c.wait()

    a = a_ref[...]
    bt = b_ref[...].T
    dsum = d_i_ref[0, :][:, None] + d_j_ref[0, :][None, :]
    sdv = jnp.sqrt(jnp.maximum(dsum - a - bt, 1e-6))
    mudv = mu_i_ref[0, :][:, None] - mu_j_ref[0, :][None, :]
    sd_up[slot] = sdv
    sd_lo[slot] = sdv.T
    mud_up[slot] = mudv
    mud_lo[slot] = -mudv.T

    for c in copies(slot, r0, c0):
        c.start()

    @pl.when(t == _P - 1)
    def _drain_tail():
        for c in copies(1 - slot, r0, c0):
            c.wait()
        for c in copies(slot, r0, c0):
            c.wait()


def kernel(mu, Sigma):
    mu2 = mu.reshape(1, _N)
    nd = _N // _BD
    diag = pl.pallas_call(
        _diag_body,
        grid=(nd,),
        in_specs=[pl.BlockSpec((_BD, _BD), lambda i: (i, i))],
        out_specs=pl.BlockSpec((1, _BD), lambda i: (0, i)),
        out_shape=jax.ShapeDtypeStruct((1, _N), jnp.float32),
    )(Sigma)

    imap = jnp.asarray(np.array([p[0] for p in _PAIRS], dtype=np.int32))
    jmap = jnp.asarray(np.array([p[1] for p in _PAIRS], dtype=np.int32))

    grid_spec = pltpu.PrefetchScalarGridSpec(
        num_scalar_prefetch=2,
        grid=(_P,),
        in_specs=[
            pl.BlockSpec((1, _B), lambda t, im, jm: (0, im[t])),
            pl.BlockSpec((1, _B), lambda t, im, jm: (0, jm[t])),
            pl.BlockSpec((1, _B), lambda t, im, jm: (0, im[t])),
            pl.BlockSpec((1, _B), lambda t, im, jm: (0, jm[t])),
            pl.BlockSpec((_B, _B), lambda t, im, jm: (im[t], jm[t])),
            pl.BlockSpec((_B, _B), lambda t, im, jm: (jm[t], im[t])),
        ],
        out_specs=[
            pl.BlockSpec(memory_space=pl.ANY),
            pl.BlockSpec(memory_space=pl.ANY),
        ],
        scratch_shapes=[
            pltpu.VMEM((2, _B, _B), jnp.float32),
            pltpu.VMEM((2, _B, _B), jnp.float32),
            pltpu.VMEM((2, _B, _B), jnp.float32),
            pltpu.VMEM((2, _B, _B), jnp.float32),
            pltpu.SemaphoreType.DMA((2,)),
        ],
    )
    mud, sd = pl.pallas_call(
        _main_body,
        grid_spec=grid_spec,
        out_shape=[
            jax.ShapeDtypeStruct((_N, _N // 128, 128), jnp.float32),
            jax.ShapeDtypeStruct((_N, _N // 128, 128), jnp.float32),
        ],
    )(imap, jmap, mu2, mu2, diag, diag, Sigma, Sigma)

    return (mud.reshape(_N * _N), sd.reshape(_N * _N))


# speedup vs baseline: 2.7495x; 1.0496x over previous
import numpy as np
import jax
import jax.numpy as jnp
from jax.experimental import pallas as pl
from jax.experimental.pallas import tpu as pltpu

_N = 4096
_BD = 128
_B = 512
_NB = _N // _B
_PAIRS = [(i, j) for i in range(_NB) for j in range(i, _NB)]
_P = len(_PAIRS)


def _diag_body(s_ref, d_ref):
    blk = s_ref[...]
    rows = jax.lax.broadcasted_iota(jnp.int32, (_BD, _BD), 0)
    cols = jax.lax.broadcasted_iota(jnp.int32, (_BD, _BD), 1)
    d_ref[0, :] = jnp.sum(jnp.where(rows == cols, blk, 0.0), axis=0)


def _main_body(im_ref, jm_ref, mu_i_ref, mu_j_ref, d_i_ref, d_j_ref,
               a_ref, b_ref, mud_hbm, sd_hbm,
               mud_up, mud_lo, sd_up, sd_lo, sems):
    t = pl.program_id(0)
    slot = jax.lax.rem(t, 2)
    ti = im_ref[t]
    tj = jm_ref[t]
    r0 = ti * _B
    c0 = tj * _B

    nch = _B // 128

    def up_copies(sl, rr, cc):
        cs = []
        for kk in range(nch):
            for src, dst in ((mud_up, mud_hbm), (sd_up, sd_hbm)):
                cs.append(pltpu.make_async_copy(
                    src.at[sl, :, pl.ds(kk * 128, 128)],
                    dst.at[pl.ds(rr, _B), cc // 128 + kk, :],
                    sems.at[sl]))
        return cs

    def lo_copies(sl, rr, cc):
        cs = []
        for kk in range(nch):
            for src, dst in ((mud_lo, mud_hbm), (sd_lo, sd_hbm)):
                cs.append(pltpu.make_async_copy(
                    src.at[sl, :, pl.ds(kk * 128, 128)],
                    dst.at[pl.ds(cc, _B), rr // 128 + kk, :],
                    sems.at[sl]))
        return cs

    def drain(step, sl):
        for c in up_copies(sl, r0, c0):
            c.wait()

        @pl.when(im_ref[step] != jm_ref[step])
        def _():
            for c in lo_copies(sl, r0, c0):
                c.wait()

    @pl.when(t >= 2)
    def _drain_prev():
        drain(t - 2, slot)

    a = a_ref[...]
    bt = b_ref[...].T
    dsum = d_i_ref[0, :][:, None] + d_j_ref[0, :][None, :]
    sdv = jnp.sqrt(jnp.maximum(dsum - a - bt, 1e-6))
    sd_up[slot] = sdv
    mud_up[slot] = mu_i_ref[0, :][:, None] - mu_j_ref[0, :][None, :]

    for c in up_copies(slot, r0, c0):
        c.start()

    @pl.when(ti != tj)
    def _mirror():
        sd_lo[slot] = sdv.T
        mud_lo[slot] = mu_j_ref[0, :][:, None] - mu_i_ref[0, :][None, :]
        for c in lo_copies(slot, r0, c0):
            c.start()

    @pl.when(t == _P - 1)
    def _drain_tail():
        drain(t - 1, 1 - slot)
        drain(t, slot)


def kernel(mu, Sigma):
    mu2 = mu.reshape(1, _N)
    nd = _N // _BD
    diag = pl.pallas_call(
        _diag_body,
        grid=(nd,),
        in_specs=[pl.BlockSpec((_BD, _BD), lambda i: (i, i))],
        out_specs=pl.BlockSpec((1, _BD), lambda i: (0, i)),
        out_shape=jax.ShapeDtypeStruct((1, _N), jnp.float32),
    )(Sigma)

    imap = jnp.asarray(np.array([p[0] for p in _PAIRS], dtype=np.int32))
    jmap = jnp.asarray(np.array([p[1] for p in _PAIRS], dtype=np.int32))

    grid_spec = pltpu.PrefetchScalarGridSpec(
        num_scalar_prefetch=2,
        grid=(_P,),
        in_specs=[
            pl.BlockSpec((1, _B), lambda t, im, jm: (0, im[t])),
            pl.BlockSpec((1, _B), lambda t, im, jm: (0, jm[t])),
            pl.BlockSpec((1, _B), lambda t, im, jm: (0, im[t])),
            pl.BlockSpec((1, _B), lambda t, im, jm: (0, jm[t])),
            pl.BlockSpec((_B, _B), lambda t, im, jm: (im[t], jm[t])),
            pl.BlockSpec((_B, _B), lambda t, im, jm: (jm[t], im[t])),
        ],
        out_specs=[
            pl.BlockSpec(memory_space=pl.ANY),
            pl.BlockSpec(memory_space=pl.ANY),
        ],
        scratch_shapes=[
            pltpu.VMEM((2, _B, _B), jnp.float32),
            pltpu.VMEM((2, _B, _B), jnp.float32),
            pltpu.VMEM((2, _B, _B), jnp.float32),
            pltpu.VMEM((2, _B, _B), jnp.float32),
            pltpu.SemaphoreType.DMA((2,)),
        ],
    )
    mud, sd = pl.pallas_call(
        _main_body,
        grid_spec=grid_spec,
        out_shape=[
            jax.ShapeDtypeStruct((_N, _N // 128, 128), jnp.float32),
            jax.ShapeDtypeStruct((_N, _N // 128, 128), jnp.float32),
        ],
    )(imap, jmap, mu2, mu2, diag, diag, Sigma, Sigma)

    return (mud.reshape(_N * _N), sd.reshape(_N * _N))
